# baseline (speedup 1.0000x reference)
import functools

import jax
import jax.numpy as jnp
from jax.experimental import pallas as pl
from jax.experimental.pallas import tpu as pltpu


def _fused_kernel(x1_ref, x2_ref,
                  w1a1_ref, w1a2_ref, w1m1_ref, w1m2_ref, b1_ref,
                  w2_ref, b2_ref,
                  wc1a_ref, wc1b_ref, bc1_ref, wc2_ref, bc2_ref,
                  o1_ref, o2_ref,
                  s1_sc, s2_sc, sum1_sc, sum2_sc, max1_sc, max2_sc, z_sc,
                  *, n_t, inv_hw, lambda_c, lambda_s):
    t = pl.program_id(1)
    C = s1_sc.shape[1]

    @pl.when(t == 0)
    def _init():
        sum1_sc[...] = jnp.zeros_like(sum1_sc)
        sum2_sc[...] = jnp.zeros_like(sum2_sc)
        max1_sc[...] = jnp.full_like(max1_sc, -jnp.inf)
        max2_sc[...] = jnp.full_like(max2_sc, -jnp.inf)

    @pl.when(t < n_t)
    def _sweep1():
        x1 = x1_ref[0]
        x2 = x2_ref[0]
        s1_sc[t] = x1
        s2_sc[t] = x2
        sum1_sc[...] += jnp.sum(x1, axis=1, keepdims=True)
        sum2_sc[...] += jnp.sum(x2, axis=1, keepdims=True)
        max1_sc[...] = jnp.maximum(max1_sc[...],
                                   jnp.max(x1, axis=1, keepdims=True))
        max2_sc[...] = jnp.maximum(max2_sc[...],
                                   jnp.max(x2, axis=1, keepdims=True))

    @pl.when(t == n_t - 1)
    def _channel_mlp():
        avg1 = sum1_sc[...] * inv_hw
        avg2 = sum2_sc[...] * inv_hw
        h = (jnp.dot(w1a1_ref[...], avg1, preferred_element_type=jnp.float32)
             + jnp.dot(w1a2_ref[...], avg2, preferred_element_type=jnp.float32)
             + jnp.dot(w1m1_ref[...], max1_sc[...],
                       preferred_element_type=jnp.float32)
             + jnp.dot(w1m2_ref[...], max2_sc[...],
                       preferred_element_type=jnp.float32)
             + b1_ref[...])
        h = jnp.maximum(h, 0.0)
        z_sc[...] = jax.nn.sigmoid(
            jnp.dot(w2_ref[...], h, preferred_element_type=jnp.float32)
            + b2_ref[...])

    @pl.when(t >= n_t)
    def _sweep2():
        tt = t - n_t
        x1 = s1_sc[tt]
        x2 = s2_sc[tt]
        hs = (jnp.dot(wc1a_ref[...], x1, preferred_element_type=jnp.float32)
              + jnp.dot(wc1b_ref[...], x2, preferred_element_type=jnp.float32)
              + bc1_ref[...])
        hs = jnp.maximum(hs, 0.0)
        s = jax.nn.sigmoid(
            jnp.dot(wc2_ref[...], hs, preferred_element_type=jnp.float32)
            + bc2_ref[...])
        s0 = s[0:1]
        s1 = s[1:2]
        z = z_sc[...]
        cw0 = z[0:C]
        cw1 = z[C:2 * C]
        o1_ref[0] = x1 + lambda_c * (cw1 * x2) + lambda_s * (s1 * x2)
        o2_ref[0] = x2 + lambda_c * (cw0 * x1) + lambda_s * (s0 * x1)


def _pick_hw_tile(hw, max_tile=1024):
    best = None
    t = 128
    while t <= min(hw, max_tile):
        if hw % t == 0:
            best = t
        t += 128
    return best if best is not None else hw


def kernel(x1, x2, w1, b1, w2, b2, wc1, bc1, wc2, bc2):
    B, C, H, W = x1.shape
    HW = H * W
    lambda_c = 0.5
    lambda_s = 0.5
    x1r = x1.reshape(B, C, HW)
    x2r = x2.reshape(B, C, HW)

    T = _pick_hw_tile(HW)
    n_t = HW // T

    hid_c = w1.shape[1]
    w1a1 = w1[0 * C:1 * C, :].T
    w1a2 = w1[1 * C:2 * C, :].T
    w1m1 = w1[2 * C:3 * C, :].T
    w1m2 = w1[3 * C:4 * C, :].T
    b1c = b1.reshape(hid_c, 1)

    w2t = w2.T
    b2c = b2.reshape(2 * C, 1)

    hid_s = wc1.shape[1]
    wc1a = wc1[0:C, :].T
    wc1b = wc1[C:2 * C, :].T
    bc1c = bc1.reshape(hid_s, 1)

    wc2t = wc2.T
    bc2c = bc2.reshape(2, 1)

    in_spec = pl.BlockSpec((1, C, T),
                           lambda b, t: (b, 0, jnp.minimum(t, n_t - 1)))
    out_spec = pl.BlockSpec((1, C, T),
                            lambda b, t: (b, 0, jnp.maximum(t - n_t, 0)))

    def const2d(shape):
        return pl.BlockSpec(shape, lambda b, t: (0, 0))

    o1, o2 = pl.pallas_call(
        functools.partial(_fused_kernel, n_t=n_t, inv_hw=1.0 / HW,
                          lambda_c=lambda_c, lambda_s=lambda_s),
        out_shape=(jax.ShapeDtypeStruct((B, C, HW), x1.dtype),
                   jax.ShapeDtypeStruct((B, C, HW), x1.dtype)),
        grid_spec=pltpu.PrefetchScalarGridSpec(
            num_scalar_prefetch=0,
            grid=(B, 2 * n_t),
            in_specs=[
                in_spec, in_spec,
                const2d((hid_c, C)), const2d((hid_c, C)),
                const2d((hid_c, C)), const2d((hid_c, C)),
                const2d((hid_c, 1)),
                const2d((2 * C, hid_c)), const2d((2 * C, 1)),
                const2d((hid_s, C)), const2d((hid_s, C)),
                const2d((hid_s, 1)),
                const2d((2, hid_s)), const2d((2, 1)),
            ],
            out_specs=[out_spec, out_spec],
            scratch_shapes=[
                pltpu.VMEM((n_t, C, T), jnp.float32),
                pltpu.VMEM((n_t, C, T), jnp.float32),
                pltpu.VMEM((C, 1), jnp.float32),
                pltpu.VMEM((C, 1), jnp.float32),
                pltpu.VMEM((C, 1), jnp.float32),
                pltpu.VMEM((C, 1), jnp.float32),
                pltpu.VMEM((2 * C, 1), jnp.float32),
            ],
        ),
        compiler_params=pltpu.CompilerParams(
            dimension_semantics=("parallel", "arbitrary")),
    )(x1r, x2r, w1a1, w1a2, w1m1, w1m2, b1c, w2t, b2c,
      wc1a, wc1b, bc1c, wc2t, bc2c)

    return o1.reshape(B, C, H, W), o2.reshape(B, C, H, W)


# speedup vs baseline: 1.1092x; 1.1092x over previous
import functools

import jax
import jax.numpy as jnp
from jax.experimental import pallas as pl
from jax.experimental.pallas import tpu as pltpu


def _fused_kernel(x1_ref, x2_ref,
                  w1a1_ref, w1a2_ref, w1m1_ref, w1m2_ref, b1_ref,
                  w2_ref, b2_ref,
                  wc1a_ref, wc1b_ref, bc1_ref, wc2_ref, bc2_ref,
                  o1_ref, o2_ref,
                  s1_sc, s2_sc, sum1_sc, sum2_sc, max1_sc, max2_sc, z_sc,
                  *, n_b, n_t, inv_hw, lambda_c, lambda_s):
    b = pl.program_id(1)
    t = pl.program_id(2)
    C = s1_sc.shape[2]
    pb = jax.lax.rem(b, 2)
    qb = 1 - pb

    @pl.when(b < n_b)
    def _sweep_in():
        @pl.when(t == 0)
        def _init():
            sum1_sc[pb] = jnp.zeros_like(sum1_sc[pb])
            sum2_sc[pb] = jnp.zeros_like(sum2_sc[pb])
            max1_sc[pb] = jnp.full_like(max1_sc[pb], -jnp.inf)
            max2_sc[pb] = jnp.full_like(max2_sc[pb], -jnp.inf)

        x1 = x1_ref[0]
        x2 = x2_ref[0]
        s1_sc[pb, t] = x1
        s2_sc[pb, t] = x2
        sum1_sc[pb] += jnp.sum(x1, axis=1, keepdims=True)
        sum2_sc[pb] += jnp.sum(x2, axis=1, keepdims=True)
        max1_sc[pb] = jnp.maximum(max1_sc[pb],
                                  jnp.max(x1, axis=1, keepdims=True))
        max2_sc[pb] = jnp.maximum(max2_sc[pb],
                                  jnp.max(x2, axis=1, keepdims=True))

        @pl.when(t == n_t - 1)
        def _channel_mlp():
            avg1 = sum1_sc[pb] * inv_hw
            avg2 = sum2_sc[pb] * inv_hw
            h = (jnp.dot(w1a1_ref[...], avg1,
                         preferred_element_type=jnp.float32)
                 + jnp.dot(w1a2_ref[...], avg2,
                           preferred_element_type=jnp.float32)
                 + jnp.dot(w1m1_ref[...], max1_sc[pb],
                           preferred_element_type=jnp.float32)
                 + jnp.dot(w1m2_ref[...], max2_sc[pb],
                           preferred_element_type=jnp.float32)
                 + b1_ref[...])
            h = jnp.maximum(h, 0.0)
            z_sc[pb] = jax.nn.sigmoid(
                jnp.dot(w2_ref[...], h, preferred_element_type=jnp.float32)
                + b2_ref[...])

    @pl.when(b > 0)
    def _sweep_out():
        x1 = s1_sc[qb, t]
        x2 = s2_sc[qb, t]
        hs = (jnp.dot(wc1a_ref[...], x1, preferred_element_type=jnp.float32)
              + jnp.dot(wc1b_ref[...], x2, preferred_element_type=jnp.float32)
              + bc1_ref[...])
        hs = jnp.maximum(hs, 0.0)
        s = jax.nn.sigmoid(
            jnp.dot(wc2_ref[...], hs, preferred_element_type=jnp.float32)
            + bc2_ref[...])
        s0 = s[0:1]
        s1 = s[1:2]
        z = z_sc[qb]
        cw0 = z[0:C]
        cw1 = z[C:2 * C]
        o1_ref[0] = x1 + lambda_c * (cw1 * x2) + lambda_s * (s1 * x2)
        o2_ref[0] = x2 + lambda_c * (cw0 * x1) + lambda_s * (s0 * x1)


def _pick_hw_tile(hw, max_tile=1024):
    best = None
    t = 128
    while t <= min(hw, max_tile):
        if hw % t == 0:
            best = t
        t += 128
    return best if best is not None else hw


def kernel(x1, x2, w1, b1, w2, b2, wc1, bc1, wc2, bc2):
    B, C, H, W = x1.shape
    HW = H * W
    lambda_c = 0.5
    lambda_s = 0.5
    x1r = x1.reshape(B, C, HW)
    x2r = x2.reshape(B, C, HW)

    T = _pick_hw_tile(HW)
    n_t = HW // T
    n_c = 2 if B % 2 == 0 else 1
    n_b = B // n_c

    hid_c = w1.shape[1]
    w1a1 = w1[0 * C:1 * C, :].T
    w1a2 = w1[1 * C:2 * C, :].T
    w1m1 = w1[2 * C:3 * C, :].T
    w1m2 = w1[3 * C:4 * C, :].T
    b1c = b1.reshape(hid_c, 1)

    w2t = w2.T
    b2c = b2.reshape(2 * C, 1)

    hid_s = wc1.shape[1]
    wc1a = wc1[0:C, :].T
    wc1b = wc1[C:2 * C, :].T
    bc1c = bc1.reshape(hid_s, 1)

    wc2t = wc2.T
    bc2c = bc2.reshape(2, 1)

    in_spec = pl.BlockSpec(
        (1, C, T),
        lambda c, b, t: (c * n_b + jnp.minimum(b, n_b - 1), 0,
                         jnp.where(b == n_b, n_t - 1, t)))
    out_spec = pl.BlockSpec(
        (1, C, T),
        lambda c, b, t: (c * n_b + jnp.maximum(b, 1) - 1, 0,
                         jnp.where(b == 0, 0, t)))

    def const2d(shape):
        return pl.BlockSpec(shape, lambda c, b, t: (0, 0))

    o1, o2 = pl.pallas_call(
        functools.partial(_fused_kernel, n_b=n_b, n_t=n_t, inv_hw=1.0 / HW,
                          lambda_c=lambda_c, lambda_s=lambda_s),
        out_shape=(jax.ShapeDtypeStruct((B, C, HW), x1.dtype),
                   jax.ShapeDtypeStruct((B, C, HW), x1.dtype)),
        grid_spec=pltpu.PrefetchScalarGridSpec(
            num_scalar_prefetch=0,
            grid=(n_c, n_b + 1, n_t),
            in_specs=[
                in_spec, in_spec,
                const2d((hid_c, C)), const2d((hid_c, C)),
                const2d((hid_c, C)), const2d((hid_c, C)),
                const2d((hid_c, 1)),
                const2d((2 * C, hid_c)), const2d((2 * C, 1)),
                const2d((hid_s, C)), const2d((hid_s, C)),
                const2d((hid_s, 1)),
                const2d((2, hid_s)), const2d((2, 1)),
            ],
            out_specs=[out_spec, out_spec],
            scratch_shapes=[
                pltpu.VMEM((2, n_t, C, T), jnp.float32),
                pltpu.VMEM((2, n_t, C, T), jnp.float32),
                pltpu.VMEM((2, C, 1), jnp.float32),
                pltpu.VMEM((2, C, 1), jnp.float32),
                pltpu.VMEM((2, C, 1), jnp.float32),
                pltpu.VMEM((2, C, 1), jnp.float32),
                pltpu.VMEM((2, 2 * C, 1), jnp.float32),
            ],
        ),
        compiler_params=pltpu.CompilerParams(
            dimension_semantics=("parallel", "arbitrary", "arbitrary")),
    )(x1r, x2r, w1a1, w1a2, w1m1, w1m2, b1c, w2t, b2c,
      wc1a, wc1b, bc1c, wc2t, bc2c)

    return o1.reshape(B, C, H, W), o2.reshape(B, C, H, W)


# speedup vs baseline: 1.1184x; 1.0083x over previous
import functools

import jax
import jax.numpy as jnp
from jax.experimental import pallas as pl
from jax.experimental.pallas import tpu as pltpu


def _fused_kernel(x1_ref, x2_ref,
                  w1a1_ref, w1a2_ref, w1m1_ref, w1m2_ref, b1_ref,
                  w2_ref, b2_ref,
                  wc1a_ref, wc1b_ref, bc1_ref, wc2_ref, bc2_ref,
                  o1_ref, o2_ref, z_sc,
                  *, T, inv_hw, lambda_c, lambda_s):
    t = pl.program_id(2)
    C = x1_ref.shape[1]

    @pl.when(t == 0)
    def _channel_mlp():
        x1 = x1_ref[0]
        x2 = x2_ref[0]
        avg1 = jnp.sum(x1, axis=1, keepdims=True) * inv_hw
        avg2 = jnp.sum(x2, axis=1, keepdims=True) * inv_hw
        max1 = jnp.max(x1, axis=1, keepdims=True)
        max2 = jnp.max(x2, axis=1, keepdims=True)
        h = (jnp.dot(w1a1_ref[...], avg1, preferred_element_type=jnp.float32)
             + jnp.dot(w1a2_ref[...], avg2, preferred_element_type=jnp.float32)
             + jnp.dot(w1m1_ref[...], max1, preferred_element_type=jnp.float32)
             + jnp.dot(w1m2_ref[...], max2, preferred_element_type=jnp.float32)
             + b1_ref[...])
        h = jnp.maximum(h, 0.0)
        z_sc[...] = jax.nn.sigmoid(
            jnp.dot(w2_ref[...], h, preferred_element_type=jnp.float32)
            + b2_ref[...])

    xs1 = x1_ref[0, :, pl.ds(t * T, T)]
    xs2 = x2_ref[0, :, pl.ds(t * T, T)]
    hs = (jnp.dot(wc1a_ref[...], xs1, preferred_element_type=jnp.float32)
          + jnp.dot(wc1b_ref[...], xs2, preferred_element_type=jnp.float32)
          + bc1_ref[...])
    hs = jnp.maximum(hs, 0.0)
    s = jax.nn.sigmoid(
        jnp.dot(wc2_ref[...], hs, preferred_element_type=jnp.float32)
        + bc2_ref[...])
    s0 = s[0:1]
    s1 = s[1:2]
    z = z_sc[...]
    cw0 = z[0:C]
    cw1 = z[C:2 * C]
    o1_ref[0] = xs1 + lambda_c * (cw1 * xs2) + lambda_s * (s1 * xs2)
    o2_ref[0] = xs2 + lambda_c * (cw0 * xs1) + lambda_s * (s0 * xs1)


def _pick_hw_tile(hw, max_tile=1024):
    best = None
    t = 128
    while t <= min(hw, max_tile):
        if hw % t == 0:
            best = t
        t += 128
    return best if best is not None else hw


def kernel(x1, x2, w1, b1, w2, b2, wc1, bc1, wc2, bc2):
    B, C, H, W = x1.shape
    HW = H * W
    lambda_c = 0.5
    lambda_s = 0.5
    x1r = x1.reshape(B, C, HW)
    x2r = x2.reshape(B, C, HW)

    T = _pick_hw_tile(HW)
    n_t = HW // T
    n_c = 2 if B % 2 == 0 else 1
    n_b = B // n_c

    hid_c = w1.shape[1]
    w1a1 = w1[0 * C:1 * C, :].T
    w1a2 = w1[1 * C:2 * C, :].T
    w1m1 = w1[2 * C:3 * C, :].T
    w1m2 = w1[3 * C:4 * C, :].T
    b1c = b1.reshape(hid_c, 1)

    w2t = w2.T
    b2c = b2.reshape(2 * C, 1)

    hid_s = wc1.shape[1]
    wc1a = wc1[0:C, :].T
    wc1b = wc1[C:2 * C, :].T
    bc1c = bc1.reshape(hid_s, 1)

    wc2t = wc2.T
    bc2c = bc2.reshape(2, 1)

    in_spec = pl.BlockSpec((1, C, HW),
                           lambda c, b, t: (c * n_b + b, 0, 0))
    out_spec = pl.BlockSpec((1, C, T),
                            lambda c, b, t: (c * n_b + b, 0, t))

    def const2d(shape):
        return pl.BlockSpec(shape, lambda c, b, t: (0, 0))

    o1, o2 = pl.pallas_call(
        functools.partial(_fused_kernel, T=T, inv_hw=1.0 / HW,
                          lambda_c=lambda_c, lambda_s=lambda_s),
        out_shape=(jax.ShapeDtypeStruct((B, C, HW), x1.dtype),
                   jax.ShapeDtypeStruct((B, C, HW), x1.dtype)),
        grid_spec=pltpu.PrefetchScalarGridSpec(
            num_scalar_prefetch=0,
            grid=(n_c, n_b, n_t),
            in_specs=[
                in_spec, in_spec,
                const2d((hid_c, C)), const2d((hid_c, C)),
                const2d((hid_c, C)), const2d((hid_c, C)),
                const2d((hid_c, 1)),
                const2d((2 * C, hid_c)), const2d((2 * C, 1)),
                const2d((hid_s, C)), const2d((hid_s, C)),
                const2d((hid_s, 1)),
                const2d((2, hid_s)), const2d((2, 1)),
            ],
            out_specs=[out_spec, out_spec],
            scratch_shapes=[
                pltpu.VMEM((2 * C, 1), jnp.float32),
            ],
        ),
        compiler_params=pltpu.CompilerParams(
            dimension_semantics=("parallel", "arbitrary", "arbitrary")),
    )(x1r, x2r, w1a1, w1a2, w1m1, w1m2, b1c, w2t, b2c,
      wc1a, wc1b, bc1c, wc2t, bc2c)

    return o1.reshape(B, C, H, W), o2.reshape(B, C, H, W)


# speedup vs baseline: 1.3850x; 1.2384x over previous
import functools

import jax
import jax.numpy as jnp
from jax.experimental import pallas as pl
from jax.experimental.pallas import tpu as pltpu


def _fused_kernel(x1_ref, x2_ref,
                  w1_ref, b1_ref, w2_ref, b2_ref,
                  wc1a_ref, wc1b_ref, bc1_ref, wc2_ref, bc2_ref,
                  o1_ref, o2_ref, *, inv_hw, lambda_c, lambda_s):
    x1 = x1_ref[0]
    x2 = x2_ref[0]
    C = x1.shape[0]
    bf16 = jnp.bfloat16

    avg1 = jnp.sum(x1, axis=1, keepdims=True) * inv_hw
    avg2 = jnp.sum(x2, axis=1, keepdims=True) * inv_hw
    max1 = jnp.max(x1, axis=1, keepdims=True)
    max2 = jnp.max(x2, axis=1, keepdims=True)
    y = jnp.concatenate([avg1, avg2, max1, max2], axis=0)
    h = (jnp.dot(w1_ref[...], y.astype(bf16),
                 preferred_element_type=jnp.float32)
         + b1_ref[...])
    h = jnp.maximum(h, 0.0)
    z = jax.nn.sigmoid(
        jnp.dot(w2_ref[...], h.astype(bf16),
                preferred_element_type=jnp.float32)
        + b2_ref[...])
    cw0 = z[0:C]
    cw1 = z[C:2 * C]

    x1b = x1.astype(bf16)
    x2b = x2.astype(bf16)
    hs = (jnp.dot(wc1a_ref[...], x1b, preferred_element_type=jnp.float32)
          + jnp.dot(wc1b_ref[...], x2b, preferred_element_type=jnp.float32)
          + bc1_ref[...])
    hs = jnp.maximum(hs, 0.0)
    s = jax.nn.sigmoid(
        jnp.dot(wc2_ref[...], hs.astype(bf16),
                preferred_element_type=jnp.float32)
        + bc2_ref[...])
    s0 = s[0:1]
    s1 = s[1:2]

    o1_ref[0] = x1 + lambda_c * (cw1 * x2) + lambda_s * (s1 * x2)
    o2_ref[0] = x2 + lambda_c * (cw0 * x1) + lambda_s * (s0 * x1)


def kernel(x1, x2, w1, b1, w2, b2, wc1, bc1, wc2, bc2):
    B, C, H, W = x1.shape
    HW = H * W
    lambda_c = 0.5
    lambda_s = 0.5
    bf16 = jnp.bfloat16
    x1r = x1.reshape(B, C, HW)
    x2r = x2.reshape(B, C, HW)

    hid_c = w1.shape[1]
    w1t = w1.T.astype(bf16)
    b1c = b1.reshape(hid_c, 1)

    w2t = w2.T.astype(bf16)
    b2c = b2.reshape(2 * C, 1)

    hid_s = wc1.shape[1]
    wc1a = wc1[0:C, :].T.astype(bf16)
    wc1b = wc1[C:2 * C, :].T.astype(bf16)
    bc1c = bc1.reshape(hid_s, 1)

    wc2t = wc2.T.astype(bf16)
    bc2c = bc2.reshape(2, 1)

    img_spec = pl.BlockSpec((1, C, HW), lambda b: (b, 0, 0))

    def const2d(shape):
        return pl.BlockSpec(shape, lambda b: (0, 0))

    o1, o2 = pl.pallas_call(
        functools.partial(_fused_kernel, inv_hw=1.0 / HW,
                          lambda_c=lambda_c, lambda_s=lambda_s),
        out_shape=(jax.ShapeDtypeStruct((B, C, HW), x1.dtype),
                   jax.ShapeDtypeStruct((B, C, HW), x1.dtype)),
        grid=(B,),
        in_specs=[
            img_spec, img_spec,
            const2d((hid_c, 4 * C)), const2d((hid_c, 1)),
            const2d((2 * C, hid_c)), const2d((2 * C, 1)),
            const2d((hid_s, C)), const2d((hid_s, C)), const2d((hid_s, 1)),
            const2d((2, hid_s)), const2d((2, 1)),
        ],
        out_specs=[img_spec, img_spec],
        compiler_params=pltpu.CompilerParams(
            dimension_semantics=("parallel",)),
    )(x1r, x2r, w1t, b1c, w2t, b2c, wc1a, wc1b, bc1c, wc2t, bc2c)

    return o1.reshape(B, C, H, W), o2.reshape(B, C, H, W)
